# Initial kernel scaffold; baseline (speedup 1.0000x reference)
import jax
import jax.numpy as jnp
from jax.experimental import pallas as pl


def kernel(edge_attr, table0, table1, table2, summary):
    raise NotImplementedError("write your pallas kernel here")


# speedup vs baseline: 1.1665x; 1.1665x over previous
import functools

import jax
import jax.numpy as jnp
from jax import lax
from jax.experimental import pallas as pl
from jax.experimental.pallas import tpu as pltpu
from jax.experimental.pallas import tpu_sc as plsc

_D = 128
_E = 320000
_T = 64
_SUM_ROW = 60
_ZERO_ROW = 61

_L = 16
_NW = 32
_PER_W = _E // _NW
_CHUNK = 400
_NCHUNK = _PER_W // _CHUNK
_GSUB = 80
_NSUB = _CHUNK // _GSUB


def _combine_body(t0_ref, t1_ref, t2_ref, su_ref, out_ref):
    def onehot(cols, sel):
        r = lax.broadcasted_iota(jnp.int32, (_T, cols), 0)
        j = lax.broadcasted_iota(jnp.int32, (_T, cols), 1)
        return ((r < 60) & (j == sel(r))).astype(jnp.float32)

    a0 = onehot(5, lambda r: r // 12)
    a1 = onehot(6, lambda r: (r // 2) % 6)
    a2 = onehot(2, lambda r: r % 2)
    rs = lax.broadcasted_iota(jnp.int32, (_T, 1), 0)
    js = lax.broadcasted_iota(jnp.int32, (_T, 1), 1)
    asu = ((rs == _SUM_ROW) & (js == 0)).astype(jnp.float32)
    out_ref[...] = (
        jnp.dot(a0, t0_ref[...], preferred_element_type=jnp.float32)
        + jnp.dot(a1, t1_ref[...], preferred_element_type=jnp.float32)
        + jnp.dot(a2, t2_ref[...], preferred_element_type=jnp.float32)
        + jnp.dot(asu, su_ref[...], preferred_element_type=jnp.float32)
    )


def _combine(table0, table1, table2, summary):
    return pl.pallas_call(
        _combine_body,
        out_shape=jax.ShapeDtypeStruct((_T, _D), jnp.float32),
    )(table0, table1, table2, summary)


def _sc_lookup(comb, c0, c1, c2):
    info = plsc.get_sparse_core_info()
    nc = info.num_cores
    mesh = plsc.VectorSubcoreMesh(core_axis_name="c", subcore_axis_name="s")

    @functools.partial(
        pl.kernel,
        out_type=jax.ShapeDtypeStruct((_E, _D), jnp.float32),
        mesh=mesh,
        scratch_types=[
            pltpu.VMEM((_CHUNK,), jnp.int32),
            pltpu.VMEM((_CHUNK,), jnp.int32),
            pltpu.VMEM((_CHUNK,), jnp.int32),
            pltpu.VMEM((_NSUB, _GSUB), jnp.int32),
            pltpu.VMEM((_CHUNK, _D), jnp.float32),
            pltpu.SemaphoreType.DMA,
        ],
    )
    def body(comb_hbm, c0_hbm, c1_hbm, c2_hbm, out_hbm,
             col0, col1, col2, idxv, rowsv, sem):
        wid = lax.axis_index("s") * nc + lax.axis_index("c")
        base = wid * _PER_W

        def chunk_body(ci, carry):
            cb = base + ci * _CHUNK
            pltpu.sync_copy(c0_hbm.at[pl.ds(cb, _CHUNK)], col0)
            pltpu.sync_copy(c1_hbm.at[pl.ds(cb, _CHUNK)], col1)
            pltpu.sync_copy(c2_hbm.at[pl.ds(cb, _CHUNK)], col2)
            for g in range(_CHUNK // _L):
                o = g * _L
                a = col0[pl.ds(o, _L)]
                b = col1[pl.ds(o, _L)]
                c = col2[pl.ds(o, _L)]
                s = a + b + c
                idx_n = (jnp.clip(a, 0, 4) * 12 + jnp.clip(b, 0, 5) * 2
                         + jnp.clip(c, 0, 1))
                idx = jnp.where(
                    s >= 0, idx_n,
                    jnp.where(s == -3,
                              jnp.full((_L,), _SUM_ROW, jnp.int32),
                              jnp.full((_L,), _ZERO_ROW, jnp.int32)))
                idxv[g // (_GSUB // _L), pl.ds((g % (_GSUB // _L)) * _L, _L)] = idx
            copies = [
                pltpu.async_copy(comb_hbm.at[idxv.at[k]],
                                 rowsv.at[pl.ds(k * _GSUB, _GSUB)], sem)
                for k in range(_NSUB)
            ]
            for cp in copies:
                cp.wait()
            pltpu.sync_copy(rowsv, out_hbm.at[pl.ds(cb, _CHUNK)])
            return carry

        lax.fori_loop(0, _NCHUNK, chunk_body, 0)

    return body(comb, c0, c1, c2)


def kernel(edge_attr, table0, table1, table2, summary):
    comb = _combine(table0, table1, table2, summary)
    ea = edge_attr.astype(jnp.int32)
    return _sc_lookup(comb, ea[:, 0], ea[:, 1], ea[:, 2])
